# Initial kernel scaffold; baseline (speedup 1.0000x reference)
import jax
import jax.numpy as jnp
from jax.experimental import pallas as pl


def kernel(x, W_out):
    raise NotImplementedError("write your pallas kernel here")


# speedup vs baseline: 35.1338x; 35.1338x over previous
import jax
import jax.numpy as jnp
from jax.experimental import pallas as pl


_C_CHUNK = 32


def _roll(v, sh, axis):
    n = v.shape[axis]
    sh = sh % n
    if sh == 0:
        return v
    axis = axis % v.ndim
    a = jax.lax.slice_in_dim(v, n - sh, n, axis=axis)
    b = jax.lax.slice_in_dim(v, 0, n - sh, axis=axis)
    return jax.lax.concatenate([a, b], axis)


def _roll2(v, di, dj):
    return _roll(_roll(v, di, -2), dj, -1)


def _gate_kernel(x_ref, g_ref):
    xb = x_ref[0]
    C, H, W = xb.shape
    row = jax.lax.broadcasted_iota(jnp.int32, (H, W), 0)
    col = jax.lax.broadcasted_iota(jnp.int32, (H, W), 1)

    zero = jnp.zeros((H, W), jnp.float32)
    S = zero
    d_row = zero
    d_col = zero
    d_dia = zero
    d_adi = zero
    m_col0 = (col == 0)[None]
    m_row0 = (row == 0)[None]
    for c0 in range(0, C, _C_CHUNK):
        xc = xb[c0:c0 + _C_CHUNK]
        S = S + jnp.sum(xc * xc, axis=0)
        r11 = _roll2(xc, 1, 1)

        d = xc - jnp.where(m_col0, r11, _roll2(xc, 0, 1))
        d_row = d_row + jnp.sum(d * d, axis=0)
        d = xc - jnp.where(m_row0, r11, _roll2(xc, 1, 0))
        d_col = d_col + jnp.sum(d * d, axis=0)
        d = xc - jnp.where(m_row0, _roll2(xc, 1, 2), r11)
        d_dia = d_dia + jnp.sum(d * d, axis=0)
        d = xc - jnp.where(m_row0, _roll2(xc, 1, 0), _roll2(xc, 1, -1))
        d_adi = d_adi + jnp.sum(d * d, axis=0)

    f2 = jnp.where(col == W - 1, _roll2(d_row, -1, -1), _roll2(d_row, 0, -1))
    f3 = jnp.where(row == H - 1, _roll2(d_col, -1, -1), _roll2(d_col, -1, 0))
    f6 = jnp.where(row == H - 1, _roll2(d_dia, -1, -2), _roll2(d_dia, -1, -1))
    f7 = jnp.where(row == H - 1, _roll2(d_adi, -1, 0), _roll2(d_adi, -1, 1))

    at00 = (row == 0) & (col == 0)
    d_row = jnp.where(at00, S, d_row)
    d_col = jnp.where(at00, S, d_col)
    d_dia = jnp.where(at00, S, d_dia)
    d_adi = jnp.where(at00, S, d_adi)
    f2 = jnp.where((row == H - 1) & (col == W - 1), S, f2)
    f3 = jnp.where((row == H - 1) & (col == W - 1), S, f3)
    f6 = jnp.where((row == H - 1) & (col == W - 2), S, f6)
    f7 = jnp.where((row == H - 1) & (col == 0), S, f7)

    def t(z):
        return jnp.tanh(jnp.sqrt(z + 1e-12))

    G = 1.6 + 0.8 * (t(d_row) + t(f2) + t(d_col) + t(f3)
                     + t(d_dia) + t(f6) + t(d_adi) + t(f7))
    g_ref[0] = G


def _out_kernel(wt_ref, x_ref, g_ref, o_ref):
    xb = x_ref[0]
    g = g_ref[0]
    z = xb * g
    o_ref[0] = xb + jnp.dot(wt_ref[...], z, preferred_element_type=jnp.float32)


def kernel(x, W_out):
    B, C, H, W = x.shape
    L = H * W

    G = pl.pallas_call(
        _gate_kernel,
        grid=(B,),
        in_specs=[pl.BlockSpec((1, C, H, W), lambda b: (b, 0, 0, 0))],
        out_specs=pl.BlockSpec((1, H, W), lambda b: (b, 0, 0)),
        out_shape=jax.ShapeDtypeStruct((B, H, W), jnp.float32),
    )(x)

    x2 = x.reshape(B, C, L)
    g3 = G.reshape(B, 1, L)
    wt = W_out.T

    Lt = 2048
    out = pl.pallas_call(
        _out_kernel,
        grid=(B, L // Lt),
        in_specs=[
            pl.BlockSpec((C, C), lambda b, l: (0, 0)),
            pl.BlockSpec((1, C, Lt), lambda b, l: (b, 0, l)),
            pl.BlockSpec((1, 1, Lt), lambda b, l: (b, 0, l)),
        ],
        out_specs=pl.BlockSpec((1, C, Lt), lambda b, l: (b, 0, l)),
        out_shape=jax.ShapeDtypeStruct((B, C, L), jnp.float32),
    )(wt, x2, g3)

    return out.reshape(B, C, H, W)


# speedup vs baseline: 89.9684x; 2.5607x over previous
import jax
import jax.numpy as jnp
from jax.experimental import pallas as pl


_C_CHUNK = 32


def _roll(v, sh, axis):
    n = v.shape[axis]
    sh = sh % n
    if sh == 0:
        return v
    axis = axis % v.ndim
    a = jax.lax.slice_in_dim(v, n - sh, n, axis=axis)
    b = jax.lax.slice_in_dim(v, 0, n - sh, axis=axis)
    return jax.lax.concatenate([a, b], axis)


def _roll2(v, di, dj):
    return _roll(_roll(v, di, -2), dj, -1)


def _fused_kernel(wt_ref, x_ref, o_ref):
    xb = x_ref[0]
    C, H, W = xb.shape
    row = jax.lax.broadcasted_iota(jnp.int32, (H, W), 0)
    col = jax.lax.broadcasted_iota(jnp.int32, (H, W), 1)

    zero = jnp.zeros((H, W), jnp.float32)
    d_row, d_col, d_dia, d_adi = zero, zero, zero, zero
    for c0 in range(0, C, _C_CHUNK):
        xc = xb[c0:c0 + _C_CHUNK]
        d = xc - _roll2(xc, 0, 1)
        d_row = d_row + jnp.sum(d * d, axis=0)
        d = xc - _roll2(xc, 1, 0)
        d_col = d_col + jnp.sum(d * d, axis=0)
        d = xc - _roll2(xc, 1, 1)
        d_dia = d_dia + jnp.sum(d * d, axis=0)
        d = xc - _roll2(xc, 1, -1)
        d_adi = d_adi + jnp.sum(d * d, axis=0)

    left = xb[:, :, 0:1]
    rightr = _roll(xb[:, :, W - 1:W], 1, 1)
    top = xb[:, 0:1, :]
    bot = xb[:, H - 1:H, :]

    d = left - rightr
    fix_row = jnp.sum(d * d, axis=0)
    d = top - _roll(bot, 1, 2)
    fix_col = jnp.sum(d * d, axis=0)
    d = top - _roll(bot, 2, 2)
    fix_dia = jnp.sum(d * d, axis=0)
    d = top - bot
    fix_adi = jnp.sum(d * d, axis=0)

    d_row = jnp.where(col == 0, fix_row, d_row)
    d_col = jnp.where(row == 0, fix_col, d_col)
    d_dia = jnp.where(row == 0, fix_dia, d_dia)
    d_adi = jnp.where(row == 0, fix_adi, d_adi)

    f2 = jnp.where(col == W - 1, _roll2(d_row, -1, -1), _roll2(d_row, 0, -1))
    f3 = jnp.where(row == H - 1, _roll2(d_col, -1, -1), _roll2(d_col, -1, 0))
    f6 = jnp.where(row == H - 1, _roll2(d_dia, -1, -2), _roll2(d_dia, -1, -1))
    f7 = jnp.where(row == H - 1, _roll2(d_adi, -1, 0), _roll2(d_adi, -1, 1))

    s_tl = jnp.sum(top[:, :, 0:1] * top[:, :, 0:1], axis=0)
    s_bot = jnp.sum(bot * bot, axis=0)
    at00 = (row == 0) & (col == 0)
    mbot = row == H - 1
    d_row = jnp.where(at00, s_tl, d_row)
    d_col = jnp.where(at00, s_tl, d_col)
    d_dia = jnp.where(at00, s_tl, d_dia)
    d_adi = jnp.where(at00, s_tl, d_adi)
    f2 = jnp.where(mbot & (col == W - 1), s_bot, f2)
    f3 = jnp.where(mbot & (col == W - 1), s_bot, f3)
    f6 = jnp.where(mbot & (col == W - 2), s_bot, f6)
    f7 = jnp.where(mbot & (col == 0), s_bot, f7)

    def t(z):
        return jnp.tanh(jnp.sqrt(z + 1e-12))

    G = 1.6 + 0.8 * (t(d_row) + t(f2) + t(d_col) + t(f3)
                     + t(d_dia) + t(f6) + t(d_adi) + t(f7))

    z2 = jnp.reshape(xb * G[None], (C, H * W))
    x2 = jnp.reshape(xb, (C, H * W))
    o2 = x2 + jnp.dot(wt_ref[...], z2, preferred_element_type=jnp.float32)
    o_ref[0] = jnp.reshape(o2, (C, H, W))


def kernel(x, W_out):
    B, C, H, W = x.shape
    wt = W_out.T

    out = pl.pallas_call(
        _fused_kernel,
        grid=(B,),
        in_specs=[
            pl.BlockSpec((C, C), lambda b: (0, 0)),
            pl.BlockSpec((1, C, H, W), lambda b: (b, 0, 0, 0)),
        ],
        out_specs=pl.BlockSpec((1, C, H, W), lambda b: (b, 0, 0, 0)),
        out_shape=jax.ShapeDtypeStruct((B, C, H, W), jnp.float32),
    )(wt, x)

    return out


# speedup vs baseline: 97.0745x; 1.0790x over previous
import jax
import jax.numpy as jnp
from jax.experimental import pallas as pl


_C_CHUNK = 32


def _roll(v, sh, axis):
    n = v.shape[axis]
    sh = sh % n
    if sh == 0:
        return v
    axis = axis % v.ndim
    a = jax.lax.slice_in_dim(v, n - sh, n, axis=axis)
    b = jax.lax.slice_in_dim(v, 0, n - sh, axis=axis)
    return jax.lax.concatenate([a, b], axis)


def _roll2(v, di, dj):
    return _roll(_roll(v, di, -2), dj, -1)


def _fused_kernel(wt_ref, x_ref, o_ref):
    xb = x_ref[0]
    C, H, W = xb.shape
    row = jax.lax.broadcasted_iota(jnp.int32, (H, W), 0)
    col = jax.lax.broadcasted_iota(jnp.int32, (H, W), 1)

    zero = jnp.zeros((H, W), jnp.float32)
    d_row, d_col, d_dia, d_adi = zero, zero, zero, zero
    for c0 in range(0, C, _C_CHUNK):
        xc = xb[c0:c0 + _C_CHUNK]
        xu = _roll(xc, 1, -2)
        d = xc - _roll(xc, 1, -1)
        d_row = d_row + jnp.sum(d * d, axis=0)
        d = xc - xu
        d_col = d_col + jnp.sum(d * d, axis=0)
        d = xc - _roll(xu, 1, -1)
        d_dia = d_dia + jnp.sum(d * d, axis=0)
        d = xc - _roll(xu, -1, -1)
        d_adi = d_adi + jnp.sum(d * d, axis=0)

    left = xb[:, :, 0:1]
    rightr = _roll(xb[:, :, W - 1:W], 1, 1)
    top = xb[:, 0:1, :]
    bot = xb[:, H - 1:H, :]

    d = left - rightr
    fix_row = jnp.sum(d * d, axis=0)
    d = top - _roll(bot, 1, 2)
    fix_col = jnp.sum(d * d, axis=0)
    d = top - _roll(bot, 2, 2)
    fix_dia = jnp.sum(d * d, axis=0)
    d = top - bot
    fix_adi = jnp.sum(d * d, axis=0)

    d_row = jnp.where(col == 0, fix_row, d_row)
    d_col = jnp.where(row == 0, fix_col, d_col)
    d_dia = jnp.where(row == 0, fix_dia, d_dia)
    d_adi = jnp.where(row == 0, fix_adi, d_adi)

    f2 = jnp.where(col == W - 1, _roll2(d_row, -1, -1), _roll2(d_row, 0, -1))
    f3 = jnp.where(row == H - 1, _roll2(d_col, -1, -1), _roll2(d_col, -1, 0))
    f6 = jnp.where(row == H - 1, _roll2(d_dia, -1, -2), _roll2(d_dia, -1, -1))
    f7 = jnp.where(row == H - 1, _roll2(d_adi, -1, 0), _roll2(d_adi, -1, 1))

    s_tl = jnp.sum(top[:, :, 0:1] * top[:, :, 0:1], axis=0)
    s_bot = jnp.sum(bot * bot, axis=0)
    at00 = (row == 0) & (col == 0)
    mbot = row == H - 1
    d_row = jnp.where(at00, s_tl, d_row)
    d_col = jnp.where(at00, s_tl, d_col)
    d_dia = jnp.where(at00, s_tl, d_dia)
    d_adi = jnp.where(at00, s_tl, d_adi)
    f2 = jnp.where(mbot & (col == W - 1), s_bot, f2)
    f3 = jnp.where(mbot & (col == W - 1), s_bot, f3)
    f6 = jnp.where(mbot & (col == W - 2), s_bot, f6)
    f7 = jnp.where(mbot & (col == 0), s_bot, f7)

    def t(z):
        return jnp.tanh(jnp.sqrt(z + 1e-12))

    G = 1.6 + 0.8 * (t(d_row) + t(f2) + t(d_col) + t(f3)
                     + t(d_dia) + t(f6) + t(d_adi) + t(f7))

    z2 = jnp.reshape(xb * G[None], (C, H * W))
    o2 = jnp.dot(wt_ref[...], z2, preferred_element_type=jnp.float32)
    o_ref[0] = xb + jnp.reshape(o2, (C, H, W))


def kernel(x, W_out):
    B, C, H, W = x.shape
    wt = W_out.T

    out = pl.pallas_call(
        _fused_kernel,
        grid=(B,),
        in_specs=[
            pl.BlockSpec((C, C), lambda b: (0, 0)),
            pl.BlockSpec((1, C, H, W), lambda b: (b, 0, 0, 0)),
        ],
        out_specs=pl.BlockSpec((1, C, H, W), lambda b: (b, 0, 0, 0)),
        out_shape=jax.ShapeDtypeStruct((B, C, H, W), jnp.float32),
    )(wt, x)

    return out
